# Initial kernel scaffold; baseline (speedup 1.0000x reference)
import jax
import jax.numpy as jnp
from jax.experimental import pallas as pl


def kernel(batch_features, embedding_weight):
    raise NotImplementedError("write your pallas kernel here")


# speedup vs baseline: 14.2465x; 14.2465x over previous
import functools

import jax
import jax.numpy as jnp
from jax import lax
from jax.experimental import pallas as pl
from jax.experimental.pallas import tpu as pltpu
from jax.experimental.pallas import tpu_sc as plsc

B, S = 1024, 50
MWL, CFD, EMB = 20, 4, 32
IDX_PER_PAIR = MWL * CFD
EXTRA = 16
FEAT = IDX_PER_PAIR + EXTRA
OUT_W = MWL * EMB + EXTRA
N = B * S

NC, NS = 2, 16
NW = NC * NS
PER_W = N // NW

C = 16
ITERS = PER_W // C
ROWS = C * IDX_PER_PAIR
GCH = ROWS // 128


def kernel(batch_features, embedding_weight):
    feats = batch_features.reshape(N, FEAT)
    mesh = plsc.VectorSubcoreMesh(core_axis_name="c", subcore_axis_name="s")

    @functools.partial(
        pl.kernel,
        mesh=mesh,
        out_type=jax.ShapeDtypeStruct((N, OUT_W), jnp.float32),
        compiler_params=pltpu.CompilerParams(use_tc_tiling_on_sc=False),
        scratch_types=[
            pltpu.VMEM((C, FEAT), jnp.float32),
            pltpu.VMEM((GCH, 128), jnp.int32),
            pltpu.VMEM((ROWS, EMB), jnp.float32),
            pltpu.VMEM((C, OUT_W), jnp.float32),
            pltpu.SemaphoreType.DMA,
        ],
    )
    def k(feats_hbm, table_hbm, out_hbm, feats_v, idx_v, rows_v, out_v, sem):
        wid = lax.axis_index("s") * NC + lax.axis_index("c")

        def body(it, carry):
            base = wid * PER_W + it * C
            pltpu.sync_copy(feats_hbm.at[pl.ds(base, C)], feats_v)

            for p in range(C):
                for kk in range(IDX_PER_PAIR // 16):
                    l = p * IDX_PER_PAIR + kk * 16
                    v = feats_v[p, pl.ds(kk * 16, 16)].astype(jnp.int32)
                    idx_v[l // 128, pl.ds(l % 128, 16)] = v

            handles = [
                pltpu.async_copy(
                    table_hbm.at[idx_v.at[j]],
                    rows_v.at[pl.ds(j * 128, 128)],
                    sem,
                )
                for j in range(GCH)
            ]
            for h in handles:
                h.wait()

            def pool(p, c2):
                rbase = p * IDX_PER_PAIR
                for g2 in range(MWL):
                    for hh in range(2):
                        cs = pl.ds(hh * 16, 16)
                        r = (
                            rows_v[rbase + g2 * 4 + 0, cs]
                            + rows_v[rbase + g2 * 4 + 1, cs]
                            + rows_v[rbase + g2 * 4 + 2, cs]
                            + rows_v[rbase + g2 * 4 + 3, cs]
                        )
                        out_v[p, pl.ds(g2 * EMB + hh * 16, 16)] = r
                out_v[p, pl.ds(MWL * EMB, 16)] = feats_v[p, pl.ds(IDX_PER_PAIR, 16)]
                return c2

            lax.fori_loop(0, C, pool, 0)

            pltpu.sync_copy(out_v, out_hbm.at[pl.ds(base, C)])
            return carry

        lax.fori_loop(0, ITERS, body, 0)

    out = k(feats, embedding_weight)
    return out.reshape(B, S, OUT_W)


# speedup vs baseline: 17.8200x; 1.2508x over previous
import functools

import jax
import jax.numpy as jnp
from jax import lax
from jax.experimental import pallas as pl
from jax.experimental.pallas import tpu as pltpu
from jax.experimental.pallas import tpu_sc as plsc

B, S = 1024, 50
MWL, CFD, EMB = 20, 4, 32
IDX_PER_PAIR = MWL * CFD
EXTRA = 16
FEAT = IDX_PER_PAIR + EXTRA
OUT_W = MWL * EMB + EXTRA
N = B * S

NC, NS = 2, 16
NW = NC * NS
PER_W = N // NW

C = 16
ITERS = PER_W // C
ROWS = C * IDX_PER_PAIR
GCH = ROWS // 128


def kernel(batch_features, embedding_weight):
    feats = batch_features.reshape(N, FEAT)
    mesh = plsc.VectorSubcoreMesh(core_axis_name="c", subcore_axis_name="s")

    @functools.partial(
        pl.kernel,
        mesh=mesh,
        out_type=jax.ShapeDtypeStruct((N, OUT_W), jnp.float32),
        compiler_params=pltpu.CompilerParams(use_tc_tiling_on_sc=False),
        scratch_types=[
            pltpu.VMEM((C, FEAT), jnp.float32),
            pltpu.VMEM((C, FEAT), jnp.float32),
            pltpu.VMEM((GCH, 128), jnp.int32),
            pltpu.VMEM((GCH, 128), jnp.int32),
            pltpu.VMEM((ROWS, EMB), jnp.float32),
            pltpu.VMEM((ROWS, EMB), jnp.float32),
            pltpu.VMEM((C, OUT_W), jnp.float32),
            pltpu.VMEM((C, OUT_W), jnp.float32),
            pltpu.SemaphoreType.DMA,
            pltpu.SemaphoreType.DMA,
            pltpu.SemaphoreType.DMA,
            pltpu.SemaphoreType.DMA,
        ],
    )
    def k(feats_hbm, table_hbm, out_hbm,
          f0, f1, i0, i1, r0, r1, o0, o1, sg0, sg1, so0, so1):
        fv, iv, rv, ov = (f0, f1), (i0, i1), (r0, r1), (o0, o1)
        sg, so = (sg0, sg1), (so0, so1)

        wid = lax.axis_index("s") * NC + lax.axis_index("c")
        wbase = wid * PER_W
        last = wbase + PER_W - C

        def fire(p, base):
            pltpu.sync_copy(feats_hbm.at[pl.ds(base, C)], fv[p])
            for pp in range(C):
                for kk in range(IDX_PER_PAIR // 16):
                    l = pp * IDX_PER_PAIR + kk * 16
                    v = fv[p][pp, pl.ds(kk * 16, 16)].astype(jnp.int32)
                    iv[p][l // 128, pl.ds(l % 128, 16)] = v
            for j in range(GCH):
                pltpu.async_copy(
                    table_hbm.at[iv[p].at[j]],
                    rv[p].at[pl.ds(j * 128, 128)],
                    sg[p],
                )

        def drain(p):
            for j in range(GCH):
                pltpu.make_async_copy(
                    table_hbm.at[iv[p].at[j]],
                    rv[p].at[pl.ds(j * 128, 128)],
                    sg[p],
                ).wait()

        def pool_and_out(p, base, t):
            @pl.when(t > 0)
            def _():
                pltpu.make_async_copy(ov[p], out_hbm.at[pl.ds(base, C)], so[p]).wait()

            def poolbody(pp, c2):
                rbase = pp * IDX_PER_PAIR
                for g2 in range(MWL):
                    for hh in range(2):
                        cs = pl.ds(hh * 16, 16)
                        r = (
                            rv[p][rbase + g2 * 4 + 0, cs]
                            + rv[p][rbase + g2 * 4 + 1, cs]
                            + rv[p][rbase + g2 * 4 + 2, cs]
                            + rv[p][rbase + g2 * 4 + 3, cs]
                        )
                        ov[p][pp, pl.ds(g2 * EMB + hh * 16, 16)] = r
                ov[p][pp, pl.ds(MWL * EMB, 16)] = fv[p][pp, pl.ds(IDX_PER_PAIR, 16)]
                return c2

            lax.fori_loop(0, C, poolbody, 0)
            pltpu.async_copy(ov[p], out_hbm.at[pl.ds(base, C)], so[p])

        fire(0, wbase)

        def body(t, carry):
            base0 = wbase + (2 * t) * C
            fire(1, base0 + C)
            drain(0)
            pool_and_out(0, base0, t)
            fire(0, lax.min(base0 + 2 * C, last))
            drain(1)
            pool_and_out(1, base0 + C, t)
            return carry

        lax.fori_loop(0, ITERS // 2, body, 0)

        drain(0)
        pltpu.make_async_copy(o0, out_hbm.at[pl.ds(wbase, C)], so0).wait()
        pltpu.make_async_copy(o1, out_hbm.at[pl.ds(wbase, C)], so1).wait()

    out = k(feats, embedding_weight)
    return out.reshape(B, S, OUT_W)
